# baseline (speedup 1.0000x reference)
import functools

import jax
import jax.numpy as jnp
import numpy as np
from jax.experimental import pallas as pl
from jax.experimental.pallas import tpu as pltpu
from jax.experimental.pallas import tpu_sc as plsc

_NUM_FEATURES = 26
_EMBED_DIM = 32
_OUTPUT_DIM = 128
_VOCAB_PER_FEATURE = 100000
_TOTAL_VOCAB = _NUM_FEATURES * _VOCAB_PER_FEATURE

_WINDOW_TOK = 64
_W = _WINDOW_TOK * _NUM_FEATURES
_LANES = 16

_PAIR_K = 2 * _NUM_FEATURES * _EMBED_DIM
_BT2 = 512


def _sc_gather(idx_flat, offs_tiled, table128):
    n = idx_flat.shape[1]
    mesh = plsc.VectorSubcoreMesh(core_axis_name="core", subcore_axis_name="subcore")

    @functools.partial(
        pl.kernel,
        out_type=jax.ShapeDtypeStruct((n, _EMBED_DIM), jnp.float32),
        mesh=mesh,
        compiler_params=pltpu.CompilerParams(use_tc_tiling_on_sc=False),
    )
    def k(idx_hbm, off_hbm, tab_hbm, out_hbm):
        tab = tab_hbm

        def body(idx_vmem, off_vmem, out_vmem):
            @pl.loop(0, _W, step=_LANES)
            def _(j):
                slc = (pl.ds(0, 1), pl.ds(j, _LANES))
                idx_vmem.at[*slc][...] = (
                    idx_vmem.at[*slc][...] + off_vmem.at[*slc][...]
                )

            pltpu.sync_copy(tab.at[idx_vmem.at[0]], out_vmem)

        pltpu.emit_pipeline(
            body,
            grid=(n // _W,),
            in_specs=[
                pl.BlockSpec((1, _W), lambda i: (0, i)),
                pl.BlockSpec((1, _W), lambda i: (0, 0)),
            ],
            out_specs=[pl.BlockSpec((_W, _EMBED_DIM), lambda i: (i, 0))],
            core_axis_name=("core", "subcore"),
            dimension_semantics=(pltpu.PARALLEL,),
        )(idx_hbm, off_hbm, out_hbm)

    return k(idx_flat, offs_tiled, table128)


def _tc_project_pairs(a, w2, b2, tp):
    nseg = _PAIR_K // 128

    def body(a_ref, w_ref, b_ref, o_ref):
        x = a_ref[...].reshape(_BT2, nseg, 128)
        acc = jnp.broadcast_to(b_ref[...], (_BT2, 2 * _OUTPUT_DIM))
        for j in range(nseg):
            e_j = x[:, j, :]
            acc = acc + jnp.dot(
                e_j, w_ref[j * 128:(j + 1) * 128, :],
                preferred_element_type=jnp.float32,
            )
        o_ref[...] = acc

    return pl.pallas_call(
        body,
        grid=(tp // _BT2,),
        in_specs=[
            pl.BlockSpec((_BT2 * nseg, 128), lambda i: (i, 0)),
            pl.BlockSpec((_PAIR_K, 2 * _OUTPUT_DIM), lambda i: (0, 0)),
            pl.BlockSpec((1, 2 * _OUTPUT_DIM), lambda i: (0, 0)),
        ],
        out_specs=pl.BlockSpec((_BT2, 2 * _OUTPUT_DIM), lambda i: (i, 0)),
        out_shape=jax.ShapeDtypeStruct((tp, 2 * _OUTPUT_DIM), jnp.float32),
    )(a, w2, b2)


_UNPAD_R = 8000


def _tc_unpad(table):
    m = table.shape[0]

    def body(x_ref, o_ref):
        y = x_ref[...].reshape(_UNPAD_R // 4, 4, _EMBED_DIM)
        o_ref[...] = jnp.concatenate(
            [y[:, 0, :], y[:, 1, :], y[:, 2, :], y[:, 3, :]], axis=1)

    return pl.pallas_call(
        body,
        grid=(m // _UNPAD_R,),
        in_specs=[pl.BlockSpec((_UNPAD_R, _EMBED_DIM), lambda i: (i, 0))],
        out_specs=pl.BlockSpec((_UNPAD_R // 4, 128), lambda i: (i, 0)),
        out_shape=jax.ShapeDtypeStruct((m // 4, 128), jnp.float32),
    )(table)


def kernel(categorical_features, embedding_table, proj_W, proj_b):
    b, l, f = categorical_features.shape
    n = b * l * f
    t = b * l
    idx = categorical_features.astype(jnp.int32).reshape(1, n)
    offs = jnp.asarray(
        np.tile(np.arange(_NUM_FEATURES, dtype=np.int32) * _VOCAB_PER_FEATURE,
                _WINDOW_TOK).reshape(1, _W)
    )
    table128 = _tc_unpad(embedding_table)
    table_rm = table128.reshape(_TOTAL_VOCAB, _EMBED_DIM)

    emb = _sc_gather(idx, offs, table_rm)

    a = emb.reshape(n * _EMBED_DIM // 128, 128)
    kdim = f * _EMBED_DIM
    w2 = jnp.zeros((_PAIR_K, 2 * _OUTPUT_DIM), jnp.float32)
    w2 = w2.at[:kdim, :_OUTPUT_DIM].set(proj_W)
    w2 = w2.at[kdim:, _OUTPUT_DIM:].set(proj_W)
    b2 = jnp.tile(proj_b, 2).reshape(1, 2 * _OUTPUT_DIM)

    out2 = _tc_project_pairs(a, w2, b2, t // 2)
    return out2.reshape(b, l, _OUTPUT_DIM)


# speedup vs baseline: 1.2133x; 1.2133x over previous
import functools

import jax
import jax.numpy as jnp
import numpy as np
from jax.experimental import pallas as pl
from jax.experimental.pallas import tpu as pltpu
from jax.experimental.pallas import tpu_sc as plsc

_NUM_FEATURES = 26
_EMBED_DIM = 32
_OUTPUT_DIM = 128
_VOCAB_PER_FEATURE = 100000
_TOTAL_VOCAB = _NUM_FEATURES * _VOCAB_PER_FEATURE

_WINDOW_TOK = 64
_W = _WINDOW_TOK * _NUM_FEATURES
_LANES = 16

_PAIR_K = 2 * _NUM_FEATURES * _EMBED_DIM
_BT2 = 512


def _window_perm() -> np.ndarray:
    q = np.arange(_W // 4)
    g, rem = q // 104, q % 104
    j, ri = rem // 8, rem % 8
    chunk = 13 * (8 * g + ri) + j
    return (4 * chunk[:, None] + np.arange(4)[None, :]).reshape(-1)


_PI4 = _window_perm()


def _sc_gather(idx_flat, offs_perm, perm, table128):
    n = idx_flat.shape[1]
    mesh = plsc.VectorSubcoreMesh(core_axis_name="core", subcore_axis_name="subcore")

    @functools.partial(
        pl.kernel,
        out_type=jax.ShapeDtypeStruct((n, _EMBED_DIM), jnp.float32),
        mesh=mesh,
        scratch_types=[pltpu.VMEM((_W,), jnp.int32)],
        compiler_params=pltpu.CompilerParams(
            use_tc_tiling_on_sc=False, needs_layout_passes=False
        ),
    )
    def k(idx_hbm, off_hbm, perm_hbm, tab_hbm, out_hbm, idx2):
        tab = tab_hbm

        def body(idx_vmem, off_vmem, perm_vmem, out_vmem):
            @pl.loop(0, _W, step=_LANES)
            def _(j):
                pv = perm_vmem.at[0, pl.ds(j, _LANES)][...]
                g = plsc.load_gather(idx_vmem.at[0], [pv])
                idx2.at[pl.ds(j, _LANES)][...] = (
                    g + off_vmem.at[0, pl.ds(j, _LANES)][...]
                )

            pltpu.sync_copy(tab.at[idx2], out_vmem)

        pltpu.emit_pipeline(
            body,
            grid=(n // _W,),
            in_specs=[
                pl.BlockSpec((1, _W), lambda i: (0, i)),
                pl.BlockSpec((1, _W), lambda i: (0, 0)),
                pl.BlockSpec((1, _W), lambda i: (0, 0)),
            ],
            out_specs=[pl.BlockSpec((_W, _EMBED_DIM), lambda i: (i, 0))],
            core_axis_name=("core", "subcore"),
            dimension_semantics=(pltpu.PARALLEL,),
        )(idx_hbm, off_hbm, perm_hbm, out_hbm)

    return k(idx_flat, offs_perm, perm, table128)


def _tc_project_pairs(a, w2, b2, tp):
    nseg = _PAIR_K // 128

    def body(a_ref, w_ref, b_ref, o_ref):
        x = a_ref[...].reshape(_BT2 // 8, nseg, 8, 128)
        acc = jnp.broadcast_to(b_ref[...], (_BT2, 2 * _OUTPUT_DIM))
        for j in range(nseg):
            e_j = x[:, j].reshape(_BT2, 128)
            acc = acc + jnp.dot(
                e_j, w_ref[j * 128:(j + 1) * 128, :],
                preferred_element_type=jnp.float32,
            )
        o_ref[...] = acc

    return pl.pallas_call(
        body,
        grid=(tp // _BT2,),
        in_specs=[
            pl.BlockSpec((_BT2 * nseg, 128), lambda i: (i, 0)),
            pl.BlockSpec((_PAIR_K, 2 * _OUTPUT_DIM), lambda i: (0, 0)),
            pl.BlockSpec((1, 2 * _OUTPUT_DIM), lambda i: (0, 0)),
        ],
        out_specs=pl.BlockSpec((_BT2, 2 * _OUTPUT_DIM), lambda i: (i, 0)),
        out_shape=jax.ShapeDtypeStruct((tp, 2 * _OUTPUT_DIM), jnp.float32),
    )(a, w2, b2)


def kernel(categorical_features, embedding_table, proj_W, proj_b):
    b, l, f = categorical_features.shape
    n = b * l * f
    t = b * l
    idx = categorical_features.astype(jnp.int32).reshape(1, n)
    offs_nat = np.tile(
        np.arange(_NUM_FEATURES, dtype=np.int32) * _VOCAB_PER_FEATURE,
        _WINDOW_TOK,
    )
    offs = jnp.asarray(offs_nat[_PI4].reshape(1, _W))
    perm = jnp.asarray(_PI4.astype(np.int32).reshape(1, _W))
    g4 = _TOTAL_VOCAB * _EMBED_DIM // 128
    table128 = embedding_table.reshape(g4, 128)
    table128 = jax.lax.optimization_barrier(table128)
    table_rm = table128.reshape(_TOTAL_VOCAB, _EMBED_DIM)

    emb = _sc_gather(idx, offs, perm, table_rm)

    a = emb.reshape(n * _EMBED_DIM // 128, 128)
    kdim = f * _EMBED_DIM
    w2 = jnp.zeros((_PAIR_K, 2 * _OUTPUT_DIM), jnp.float32)
    w2 = w2.at[:kdim, :_OUTPUT_DIM].set(proj_W)
    w2 = w2.at[kdim:, _OUTPUT_DIM:].set(proj_W)
    b2 = jnp.tile(proj_b, 2).reshape(1, 2 * _OUTPUT_DIM)

    out2 = _tc_project_pairs(a, w2, b2, t // 2)
    return out2.reshape(b, l, _OUTPUT_DIM)
